# baseline (speedup 1.0000x reference)
import functools

import jax
import jax.numpy as jnp
import numpy as np
from jax import lax
from jax.experimental import pallas as pl
from jax.experimental.pallas import tpu as pltpu
from jax.experimental.pallas import tpu_sc as plsc

B = 4096
NUM_FIELDS = 26
VOCAB = 100000
EMB = 64
NUM_DENSE = 13
NV = NUM_FIELDS + 1
INTER = NV * (NV - 1) // 2

NC, NS = 2, 16
NW = NC * NS
TOTAL_ROWS = B * NUM_FIELDS
ROWS_PER_W = TOTAL_ROWS // NW
CHUNK = 128
NCHUNK = ROWS_PER_W // CHUNK

_tril_i, _tril_j = np.tril_indices(NV, -1)
_pair_map = np.full((NV * NV,), INTER, dtype=np.int32)
_pair_map[_tril_i * NV + _tril_j] = np.arange(INTER, dtype=np.int32)


def _sc_gather(table, idx3):
  mesh = plsc.VectorSubcoreMesh(core_axis_name="c", subcore_axis_name="s")

  @functools.partial(
      pl.kernel,
      mesh=mesh,
      out_type=jax.ShapeDtypeStruct((TOTAL_ROWS, EMB), jnp.float32),
      scratch_types=[
          pltpu.VMEM((NCHUNK, CHUNK), jnp.int32),
          pltpu.VMEM((CHUNK, EMB), jnp.float32),
          pltpu.SemaphoreType.DMA,
      ],
      compiler_params=pltpu.CompilerParams(use_tc_tiling_on_sc=False),
  )
  def k(table_hbm, idx_hbm, out_hbm, idx_v, rows_v, sem):
    wid = lax.axis_index("s") * NC + lax.axis_index("c")
    base = wid * ROWS_PER_W
    pltpu.sync_copy(idx_hbm.at[wid], idx_v)

    def body(c, _):
      pltpu.async_copy(table_hbm.at[idx_v.at[c]], rows_v, sem).wait()
      pltpu.sync_copy(rows_v, out_hbm.at[pl.ds(base + c * CHUNK, CHUNK)])
      return 0

    lax.fori_loop(0, NCHUNK, body, 0)

  return k(table, idx3)


def _tc_body(num_ref, emb_ref, bw0, bb0, bw1, bb1, bw2, bb2,
             tw0a, wz, tb0, tw1, tb1, tw2, tb2, tw3, tb3, tw4, tb4,
             out_ref):
  f32 = jnp.float32
  x = num_ref[...]
  h = jnp.maximum(jnp.dot(x, bw0[...], preferred_element_type=f32) + bb0[...], 0.0)
  h = jnp.maximum(jnp.dot(h, bw1[...], preferred_element_type=f32) + bb1[...], 0.0)
  bmo = jnp.maximum(jnp.dot(h, bw2[...], preferred_element_type=f32) + bb2[...], 0.0)
  t3 = jnp.concatenate([bmo[:, None, :], emb_ref[...]], axis=1)
  z3 = lax.dot_general(t3, t3, (((2,), (2,)), ((0,), (0,))),
                       preferred_element_type=f32)
  zf = z3.reshape(z3.shape[0], NV * NV)
  x1 = jnp.maximum(jnp.dot(bmo, tw0a[...], preferred_element_type=f32)
                   + jnp.dot(zf, wz[...], preferred_element_type=f32)
                   + tb0[...], 0.0)
  x2 = jnp.maximum(jnp.dot(x1, tw1[...], preferred_element_type=f32) + tb1[...], 0.0)
  x3 = jnp.maximum(jnp.dot(x2, tw2[...], preferred_element_type=f32) + tb2[...], 0.0)
  x4 = jnp.maximum(jnp.dot(x3, tw3[...], preferred_element_type=f32) + tb3[...], 0.0)
  out_ref[...] = jnp.dot(x4, tw4[...], preferred_element_type=f32) + tb4[...]


def kernel(numerical_input, categorical_inputs, emb_table,
           bw0, bb0, bw1, bb1, bw2, bb2,
           tw0, tb0, tw1, tb1, tw2, tb2, tw3, tb3, tw4, tb4):
  offsets = (jnp.arange(NUM_FIELDS, dtype=jnp.int32) * VOCAB)[None, :]
  idx3 = (categorical_inputs + offsets).reshape(NW, NCHUNK, CHUNK)
  tw0_pad = jnp.concatenate([tw0[EMB:], jnp.zeros((1, tw0.shape[1]), tw0.dtype)], axis=0)
  wz = jnp.take(tw0_pad, jnp.asarray(_pair_map), axis=0)
  tw0a = tw0[:EMB]

  emb_rows = _sc_gather(emb_table, idx3)
  emb3 = emb_rows.reshape(B, NUM_FIELDS, EMB)

  bm = 256
  grid = (B // bm,)
  full = lambda shape: pl.BlockSpec(shape, lambda i: (0,) * len(shape))
  out = pl.pallas_call(
      _tc_body,
      grid=grid,
      in_specs=[
          pl.BlockSpec((bm, NUM_DENSE), lambda i: (i, 0)),
          pl.BlockSpec((bm, NUM_FIELDS, EMB), lambda i: (i, 0, 0)),
          full((NUM_DENSE, 512)), full((1, 512)),
          full((512, 256)), full((1, 256)),
          full((256, EMB)), full((1, EMB)),
          full((EMB, 1024)), full((NV * NV, 1024)), full((1, 1024)),
          full((1024, 1024)), full((1, 1024)),
          full((1024, 512)), full((1, 512)),
          full((512, 256)), full((1, 256)),
          full((256, 1)), full((1, 1)),
      ],
      out_specs=pl.BlockSpec((bm, 1), lambda i: (i, 0)),
      out_shape=jax.ShapeDtypeStruct((B, 1), jnp.float32),
  )(
      numerical_input, emb3,
      bw0, bb0[None, :], bw1, bb1[None, :], bw2, bb2[None, :],
      tw0a, wz, tb0[None, :],
      tw1, tb1[None, :], tw2, tb2[None, :], tw3, tb3[None, :],
      tw4, tb4[None, :],
  )
  return out
